# baseline (speedup 1.0000x reference)
import functools

import jax
import jax.numpy as jnp
from jax import lax
from jax.experimental import pallas as pl
from jax.experimental.pallas import tpu as pltpu
from jax.experimental.pallas import tpu_sc as plsc


def _stats_body(lref, sref, cref, nacc, *, nc, rows, w, h, nchunk):
    ncls = nc - 1
    j = pl.program_id(1)

    x = lref[0]

    best = x[0]
    idx = jnp.zeros((rows, w), jnp.int32)
    for c in range(1, nc):
        v = x[c]
        m = v > best
        best = jnp.where(m, v, best)
        idx = jnp.where(m, c, idx)

    colp = jax.lax.broadcasted_iota(
        jnp.int32, (rows, w), 1).astype(jnp.float32) + 1.0
    colr = jnp.float32(w + 1) - colp
    ones_row = jnp.ones((1, rows), jnp.float32)
    mn_l, mx_l, nparts = [], [], []
    for c in range(1, nc):
        mf = jnp.where(idx == c, 1.0, 0.0)
        mx_l.append(jnp.max(mf * colp, axis=1, keepdims=True))
        mn_l.append(jnp.max(mf * colr, axis=1, keepdims=True))
        part = jax.lax.dot_general(
            ones_row, mf, (((1,), (0,)), ((), ())),
            preferred_element_type=jnp.float32)
        nparts.append(jnp.sum(part, axis=1, keepdims=True))
    mn = w - jnp.concatenate(mn_l, axis=1).astype(jnp.int32)
    mx = jnp.concatenate(mx_l, axis=1).astype(jnp.int32) - 1
    nchunkcnt = jnp.concatenate(nparts, axis=1).astype(jnp.int32)

    pk = mn * 1024 + (mx + 1)
    pkt = jnp.transpose(pk.astype(jnp.float32)).astype(jnp.int32)
    sref[0, :ncls, :] = pkt

    @pl.when(j == 0)
    def _():
        nacc[:1, :] = jnp.zeros((1, 128), jnp.int32)

    nacc[:1, :ncls] = nacc[:1, :ncls] + nchunkcnt

    @pl.when(j == nchunk - 1)
    def _():
        nt = jnp.transpose(nacc[:1, :32].astype(jnp.float32))
        cref[0] = jnp.broadcast_to(nt, (32, 16)).astype(jnp.int32)


def _merge_body(stats_hbm, counts_hbm, out_hbm, stats_v, counts_v, out_v,
                *, h, w, ncls, nbatch):
    wid = lax.axis_index("s") * 2 + lax.axis_index("c")
    ntask = nbatch * ncls
    ngrp = h // 16
    lane = jnp.arange(16, dtype=jnp.int32)

    def do_task(t):
        b = t // ncls
        c = t % ncls
        pltpu.sync_copy(stats_hbm.at[b, c], stats_v)
        pltpu.sync_copy(counts_hbm.at[b, c], counts_v)

        shift_idx = jnp.maximum(lane - 1, 0)

        def grp(g, carry):
            run, acc_s = carry
            rows16 = g * 16 + lane
            pk = stats_v[pl.ds(g * 16, 16)]
            mnv = pk >> 10
            mxv = (pk & 1023) - 1
            occ = mxv >= 0
            firstv = mnv - rows16
            lastv = mxv - rows16
            pkm = jnp.where(occ, (rows16 + 1) * 2048 + (lastv + h), -1)
            im = plsc.cummax(pkm)
            sh = lax.gather(
                im, shift_idx[:, None],
                dimension_numbers=lax.GatherDimensionNumbers(
                    offset_dims=(), collapsed_slice_dims=(0,),
                    start_index_map=(0,)),
                slice_sizes=(1,),
                mode=lax.GatherScatterMode.PROMISE_IN_BOUNDS)
            excl = jnp.where(lane == 0, -1, sh)
            erun = jnp.maximum(excl, jnp.full((16,), run, jnp.int32))
            prev_ok = erun >= 0
            prev_last = (erun & 2047) - h
            cross = jnp.where(occ & prev_ok, jnp.abs(firstv - prev_last), 0)
            srow = jnp.where(occ, mxv - mnv, 0)
            run = jnp.maximum(run, jnp.max(pkm))
            return run, acc_s + cross + srow

        run0 = jnp.int32(-1)
        _, acc_s = lax.fori_loop(0, ngrp, grp,
                                 (run0, jnp.zeros((16,), jnp.int32)))
        s_vec = jnp.full((16,), jnp.sum(acc_s)).astype(jnp.float32)
        n_vec = counts_v[...].astype(jnp.float32)
        mean = s_vec / jnp.maximum(n_vec - 1.0, 1.0)
        contrib = jnp.where(n_vec >= 2.0, mean / (n_vec + 1.0), 0.0)
        out_v[...] = jnp.where(lane == 0, contrib, 0.0)
        pltpu.sync_copy(out_v, out_hbm.at[t])

    do_task(wid)

    @pl.when(wid < ntask - 32)
    def _():
        do_task(wid + 32)


def kernel(logits, labels):
    del labels
    bs, nc, h, w = logits.shape
    ncls = nc - 1
    rows = 256
    nchunk = h // rows

    body = functools.partial(_stats_body, nc=nc, rows=rows, w=w, h=h,
                             nchunk=nchunk)
    stats, counts = pl.pallas_call(
        body,
        grid=(bs, nchunk),
        in_specs=[
            pl.BlockSpec((1, nc, rows, w), lambda b, j: (b, 0, j, 0)),
        ],
        out_specs=[
            pl.BlockSpec((1, 32, rows), lambda b, j: (b, 0, j)),
            pl.BlockSpec((1, 32, 16), lambda b, j: (b, 0, 0)),
        ],
        out_shape=[
            jax.ShapeDtypeStruct((bs, 32, h), jnp.int32),
            jax.ShapeDtypeStruct((bs, 32, 16), jnp.int32),
        ],
        scratch_shapes=[
            pltpu.VMEM((8, 128), jnp.int32),
        ],
        compiler_params=pltpu.CompilerParams(
            dimension_semantics=("arbitrary", "arbitrary"),
        ),
    )(logits)

    mesh = plsc.VectorSubcoreMesh(core_axis_name="c", subcore_axis_name="s")
    merge = functools.partial(_merge_body, h=h, w=w, ncls=ncls, nbatch=bs)
    out = pl.kernel(
        merge,
        out_type=jax.ShapeDtypeStruct((bs * ncls, 16), jnp.float32),
        mesh=mesh,
        scratch_types=[
            pltpu.VMEM((h,), jnp.int32),
            pltpu.VMEM((16,), jnp.int32),
            pltpu.VMEM((16,), jnp.float32),
        ],
        compiler_params=pltpu.CompilerParams(needs_layout_passes=False),
    )(stats, counts)
    return jnp.sum(out)
